# baseline (speedup 1.0000x reference)
import dataclasses

import jax
import jax.numpy as jnp
from jax import lax
from jax.experimental import pallas as pl
from jax.experimental.pallas import tpu as pltpu
from jax.experimental.pallas import tpu_sc as plsc

BATCH = 16384
DIM = 32
LANES = 16
ROWS_PER_LINE = 4
NUM_CORES = 2
NUM_SUBCORES = 16
NUM_WORKERS = NUM_CORES * NUM_SUBCORES
BPW = BATCH // NUM_WORKERS
CHUNK = 256

TC_BLOCK = 2048

TR_LANES = 32768
TR_GRID = -(-1000000 // TR_LANES)
QUARTER = TR_LANES // 4
N_LINES = TR_GRID * QUARTER


def _relayout_body(in_ref, out_ref):
    st = jnp.concatenate(
        [in_ref[:, QUARTER * a: QUARTER * (a + 1)]
         for a in range(ROWS_PER_LINE)], axis=0)
    out_ref[...] = jnp.swapaxes(st, 0, 1)


def _tc_relayout(tab_t):
    return pl.pallas_call(
        _relayout_body,
        out_shape=jax.ShapeDtypeStruct((N_LINES, 128), jnp.float32),
        grid=(TR_GRID,),
        in_specs=[pl.BlockSpec((DIM, TR_LANES), lambda i: (0, i))],
        out_specs=pl.BlockSpec((TR_LANES // 4, 128), lambda i: (i, 0)),
        compiler_params=pltpu.CompilerParams(
            dimension_semantics=("parallel",)),
    )(tab_t)


def _line_of(idx):
    return lax.bitwise_or(
        lax.shift_left(lax.shift_right_logical(idx, 15), 13),
        lax.bitwise_and(idx, QUARTER - 1))


def _gather_body(idx_hbm, tab_hbm, out_hbm, idx_v, ridx_v, buf0, buf1, sem0,
                 sem1):
    wid = lax.axis_index("s") * NUM_CORES + lax.axis_index("c")
    base = wid * BPW

    pltpu.sync_copy(idx_hbm.at[pl.ds(base, BPW)], idx_v)

    @pl.loop(0, BPW, step=LANES)
    def _(i0):
        ridx_v[pl.ds(i0, LANES)] = _line_of(idx_v[pl.ds(i0, LANES)])

    c0 = pltpu.async_copy(tab_hbm.at[ridx_v.at[pl.ds(0, CHUNK)]], buf0, sem0)
    c1 = pltpu.async_copy(
        tab_hbm.at[ridx_v.at[pl.ds(CHUNK, CHUNK)]], buf1, sem1)
    c0.wait()
    pltpu.sync_copy(buf0, out_hbm.at[pl.ds(base, CHUNK), :])
    c1.wait()
    pltpu.sync_copy(buf1, out_hbm.at[pl.ds(base + CHUNK, CHUNK), :])


def _sc_gather(idx, tab):
    mesh = plsc.VectorSubcoreMesh(core_axis_name="c", subcore_axis_name="s")
    cp = dataclasses.replace(
        pltpu.CompilerParams(),
        needs_layout_passes=False,
        use_tc_tiling_on_sc=True,
    )
    run = pl.kernel(
        _gather_body,
        out_type=jax.ShapeDtypeStruct((BATCH, 128), jnp.float32),
        mesh=mesh,
        scratch_types=[
            pltpu.VMEM((BPW,), jnp.int32),
            pltpu.VMEM((BPW,), jnp.int32),
            pltpu.VMEM((CHUNK, 128), jnp.float32),
            pltpu.VMEM((CHUNK, 128), jnp.float32),
            pltpu.SemaphoreType.DMA,
            pltpu.SemaphoreType.DMA,
        ],
        compiler_params=cp,
    )
    return run(idx, tab)


def _finish_body(u_ref, v_ref, gu_ref, gv_ref, out_ref):
    def select(rows, idx_col):
        g = lax.shift_right_logical(idx_col, 13) % ROWS_PER_LINE
        half = jnp.where(g >= 2, rows[:, 2 * DIM:4 * DIM],
                         rows[:, 0:2 * DIM])
        return jnp.where(g % 2 == 1, half[:, DIM:2 * DIM], half[:, 0:DIM])

    usel = select(u_ref[...], gu_ref[...])
    vsel = select(v_ref[...], gv_ref[...])
    dots = jnp.sum(usel * vsel, axis=1, keepdims=True)
    out_ref[...] = 1.0 / (1.0 + jnp.exp(-dots))


def _tc_finish(urows, irows, users_col, items_col):
    return pl.pallas_call(
        _finish_body,
        out_shape=jax.ShapeDtypeStruct((BATCH, 1), jnp.float32),
        grid=(BATCH // TC_BLOCK,),
        in_specs=[
            pl.BlockSpec((TC_BLOCK, 128), lambda i: (i, 0)),
            pl.BlockSpec((TC_BLOCK, 128), lambda i: (i, 0)),
            pl.BlockSpec((TC_BLOCK, 1), lambda i: (i, 0)),
            pl.BlockSpec((TC_BLOCK, 1), lambda i: (i, 0)),
        ],
        out_specs=pl.BlockSpec((TC_BLOCK, 1), lambda i: (i, 0)),
    )(urows, irows, users_col, items_col)


@jax.jit
def kernel(users, items, user_table, item_table):
    ut = _tc_relayout(user_table.T)
    urows = _sc_gather(users, ut)
    it = _tc_relayout(item_table.T)
    irows = _sc_gather(items, it)
    out = _tc_finish(urows, irows,
                     users.reshape(BATCH, 1), items.reshape(BATCH, 1))
    return out.reshape(BATCH)
